# Initial kernel scaffold; baseline (speedup 1.0000x reference)
import jax
import jax.numpy as jnp
from jax.experimental import pallas as pl


def kernel(x, edge_index, W1, al1, ar1, b1, W2, al2, ar2, b2):
    raise NotImplementedError("write your pallas kernel here")


# speedup vs baseline: 55.7419x; 55.7419x over previous
import functools

import jax
import jax.numpy as jnp
from jax import lax
from jax.experimental import pallas as pl
from jax.experimental.pallas import tpu as pltpu
from jax.experimental.pallas import tpu_sc as plsc

N = 10000
NPAD = 10240
E = 320000
D = 128

NC = 2
NS = 16
NW = NC * NS
EPT = E // NW
B = 80
NB = EPT // B
NBUF = 4
RPT = NPAD // NS
RBLK = 2048
GRID = NPAD // RBLK


def _proj_body(x_ref, w_ref, al_ref, ar_ref, h_ref, el_ref, er_ref):
    h = jnp.dot(x_ref[...], w_ref[...], preferred_element_type=jnp.float32)
    h_ref[...] = h
    el_ref[...] = jnp.sum(h * al_ref[...], axis=1)[None, :]
    er_ref[...] = jnp.sum(h * ar_ref[...], axis=1)[None, :]


_proj = pl.pallas_call(
    _proj_body,
    grid=(GRID,),
    in_specs=[
        pl.BlockSpec((RBLK, D), lambda i: (i, 0)),
        pl.BlockSpec((D, D), lambda i: (0, 0)),
        pl.BlockSpec((1, D), lambda i: (0, 0)),
        pl.BlockSpec((1, D), lambda i: (0, 0)),
    ],
    out_specs=[
        pl.BlockSpec((RBLK, D), lambda i: (i, 0)),
        pl.BlockSpec((1, RBLK), lambda i: (0, i)),
        pl.BlockSpec((1, RBLK), lambda i: (0, i)),
    ],
    out_shape=[
        jax.ShapeDtypeStruct((NPAD, D), jnp.float32),
        jax.ShapeDtypeStruct((1, NPAD), jnp.float32),
        jax.ShapeDtypeStruct((1, NPAD), jnp.float32),
    ],
)


def _mid_body(acc_ref, den_ref, b_ref, w_ref, al_ref, ar_ref,
              h_ref, el_ref, er_ref):
    acc = acc_ref[0] + acc_ref[1]
    den = den_ref[0] + den_ref[1]
    hin = acc / (den[:, None] + 1e-9) + b_ref[...]
    hin = jnp.maximum(hin, 0.0)
    h = jnp.dot(hin, w_ref[...], preferred_element_type=jnp.float32)
    h_ref[...] = h
    el_ref[...] = jnp.sum(h * al_ref[...], axis=1)[None, :]
    er_ref[...] = jnp.sum(h * ar_ref[...], axis=1)[None, :]


_mid = pl.pallas_call(
    _mid_body,
    grid=(GRID,),
    in_specs=[
        pl.BlockSpec((NC, RBLK, D), lambda i: (0, i, 0)),
        pl.BlockSpec((NC, RBLK), lambda i: (0, i)),
        pl.BlockSpec((1, D), lambda i: (0, 0)),
        pl.BlockSpec((D, D), lambda i: (0, 0)),
        pl.BlockSpec((1, D), lambda i: (0, 0)),
        pl.BlockSpec((1, D), lambda i: (0, 0)),
    ],
    out_specs=[
        pl.BlockSpec((RBLK, D), lambda i: (i, 0)),
        pl.BlockSpec((1, RBLK), lambda i: (0, i)),
        pl.BlockSpec((1, RBLK), lambda i: (0, i)),
    ],
    out_shape=[
        jax.ShapeDtypeStruct((NPAD, D), jnp.float32),
        jax.ShapeDtypeStruct((1, NPAD), jnp.float32),
        jax.ShapeDtypeStruct((1, NPAD), jnp.float32),
    ],
)


def _fin_body(acc_ref, den_ref, b_ref, o_ref):
    acc = acc_ref[0] + acc_ref[1]
    den = den_ref[0] + den_ref[1]
    o_ref[...] = acc / (den[:, None] + 1e-9) + b_ref[...]


_fin = pl.pallas_call(
    _fin_body,
    grid=(GRID,),
    in_specs=[
        pl.BlockSpec((NC, RBLK, D), lambda i: (0, i, 0)),
        pl.BlockSpec((NC, RBLK), lambda i: (0, i)),
        pl.BlockSpec((1, D), lambda i: (0, 0)),
    ],
    out_specs=pl.BlockSpec((RBLK, D), lambda i: (i, 0)),
    out_shape=jax.ShapeDtypeStruct((NPAD, D), jnp.float32),
)


_mesh = plsc.VectorSubcoreMesh(
    core_axis_name="c", subcore_axis_name="s", num_cores=NC, num_subcores=NS)


@functools.partial(
    pl.kernel,
    out_type=[
        jax.ShapeDtypeStruct((NC * NPAD, D), jnp.float32),
        jax.ShapeDtypeStruct((NC * NPAD,), jnp.float32),
    ],
    mesh=_mesh,
    compiler_params=pltpu.CompilerParams(needs_layout_passes=False),
    scratch_types=(
        [pltpu.VMEM((B,), jnp.int32) for _ in range(NBUF)]
        + [pltpu.VMEM((B,), jnp.int32) for _ in range(NBUF)]
        + [pltpu.VMEM((B,), jnp.float32) for _ in range(NBUF)]
        + [pltpu.VMEM((B,), jnp.float32) for _ in range(NBUF)]
        + [pltpu.VMEM((B,), jnp.float32) for _ in range(NBUF)]
        + [pltpu.VMEM((B, D), jnp.float32) for _ in range(NBUF)]
        + [
            pltpu.VMEM((RPT,), jnp.float32),
            pltpu.VMEM_SHARED((NPAD, D), jnp.float32),
            pltpu.VMEM_SHARED((NPAD,), jnp.float32),
        ]
        + [pltpu.SemaphoreType.DMA for _ in range(3 * NBUF)]
    ),
)
def _edge(h_hbm, el_hbm, er_hbm, src_hbm, dst_hbm, acc_out, den_out,
          *scratch):
    srcs = scratch[0:NBUF]
    dsts = scratch[NBUF:2 * NBUF]
    elbs = scratch[2 * NBUF:3 * NBUF]
    erbs = scratch[3 * NBUF:4 * NBUF]
    ees = scratch[4 * NBUF:5 * NBUF]
    rows = scratch[5 * NBUF:6 * NBUF]
    zer_v = scratch[6 * NBUF]
    acc_sh = scratch[6 * NBUF + 1]
    den_sh = scratch[6 * NBUF + 2]
    semA = scratch[6 * NBUF + 3:6 * NBUF + 3 + NBUF]
    semR = scratch[6 * NBUF + 3 + NBUF:6 * NBUF + 3 + 2 * NBUF]
    semS = scratch[6 * NBUF + 3 + 2 * NBUF:6 * NBUF + 3 + 3 * NBUF]

    c = lax.axis_index("c")
    s = lax.axis_index("s")
    wid = c * NS + s
    ebase = wid * EPT

    zero16 = jnp.zeros((16,), jnp.float32)

    def _zrow(r, carry):
        for j in range(D // 16):
            rows[0][r, pl.ds(j * 16, 16)] = zero16
        return carry

    lax.fori_loop(0, B, _zrow, 0)

    def _zv(i, carry):
        zer_v[pl.ds(i * 16, 16)] = zero16
        return carry

    lax.fori_loop(0, RPT // 16, _zv, 0)

    def _zacc(i, carry):
        pltpu.sync_copy(rows[0], acc_sh.at[pl.ds(s * RPT + i * B, B)])
        return carry

    lax.fori_loop(0, RPT // B, _zacc, 0)
    pltpu.sync_copy(zer_v, den_sh.at[pl.ds(s * RPT, RPT)])
    plsc.subcore_barrier()

    def _issue_idx(b, p):
        off = ebase + lax.rem(b, NB) * B
        pltpu.async_copy(src_hbm.at[pl.ds(off, B)], srcs[p], semA[p])
        pltpu.async_copy(dst_hbm.at[pl.ds(off, B)], dsts[p], semA[p])

    def _wait_idx(p):
        pltpu.make_async_copy(
            src_hbm.at[pl.ds(ebase, B)], srcs[p], semA[p]).wait()
        pltpu.make_async_copy(
            dst_hbm.at[pl.ds(ebase, B)], dsts[p], semA[p]).wait()

    def _issue_gathers(p):
        pltpu.async_copy(h_hbm.at[srcs[p]], rows[p], semR[p])
        pltpu.async_copy(el_hbm.at[srcs[p]], elbs[p], semA[p])
        pltpu.async_copy(er_hbm.at[dsts[p]], erbs[p], semA[p])

    def _wait_meta(p):
        pltpu.make_async_copy(
            el_hbm.at[srcs[p]], elbs[p], semA[p]).wait()
        pltpu.make_async_copy(
            er_hbm.at[dsts[p]], erbs[p], semA[p]).wait()

    def _wait_rows(p):
        pltpu.make_async_copy(h_hbm.at[srcs[p]], rows[p], semR[p]).wait()

    def _wait_scat(p):
        pltpu.make_async_copy(ees[p], den_sh.at[dsts[p]], semS[p]).wait()
        pltpu.make_async_copy(rows[p], acc_sh.at[dsts[p]], semS[p]).wait()

    def _refill(b, p):
        _wait_scat(p)
        _issue_idx(b, p)
        _wait_idx(p)
        _issue_gathers(p)

    def _process(p):
        _wait_meta(p)

        def _grp(g, carry2):
            ev = elbs[p][pl.ds(g * 16, 16)] + erbs[p][pl.ds(g * 16, 16)]
            ev = jnp.where(ev >= 0.0, ev, ev * jnp.float32(0.2))
            ees[p][pl.ds(g * 16, 16)] = jnp.exp(ev)
            return carry2

        lax.fori_loop(0, B // 16, _grp, 0)
        pltpu.async_copy(ees[p], den_sh.at[dsts[p]], semS[p], add=True)
        _wait_rows(p)

        def _scale(g, carry2):
            ee16 = ees[p][pl.ds(g * 16, 16)]
            for k in range(16):
                sc = ee16[k]
                r = g * 16 + k
                for j in range(D // 16):
                    rows[p][r, pl.ds(j * 16, 16)] = (
                        rows[p][r, pl.ds(j * 16, 16)] * sc)
            return carry2

        lax.fori_loop(0, B // 16, _scale, 0)
        pltpu.async_copy(rows[p], acc_sh.at[dsts[p]], semS[p], add=True)

    for p in range(NBUF):
        _issue_idx(p, p)
    for p in range(NBUF):
        _wait_idx(p)
        _issue_gathers(p)

    @pl.loop(0, NB - 1, step=NBUF)
    def _body(g):
        for k in range(NBUF):
            _process(k)
            if k >= 1:
                _refill(g + (k - 1) + NBUF, k - 1)
        _refill(g + (NBUF - 1) + NBUF, NBUF - 1)

    _process(0)
    _wait_scat(0)
    for p in range(1, NBUF):
        _wait_meta(p)
        _wait_rows(p)
    plsc.subcore_barrier()

    obase = c * NPAD + s * RPT
    pltpu.sync_copy(acc_sh.at[pl.ds(s * RPT, RPT)],
                    acc_out.at[pl.ds(obase, RPT)])
    pltpu.sync_copy(den_sh.at[pl.ds(s * RPT, RPT)],
                    den_out.at[pl.ds(obase, RPT)])


def kernel(x, edge_index, W1, al1, ar1, b1, W2, al2, ar2, b2):
    src = edge_index[0].astype(jnp.int32)
    dst = edge_index[1].astype(jnp.int32)
    xp = jnp.pad(x, ((0, NPAD - N), (0, 0)))

    h1, el1, er1 = _proj(xp, W1, al1.reshape(1, D), ar1.reshape(1, D))
    acc1, den1 = _edge(h1, el1.reshape(NPAD), er1.reshape(NPAD), src, dst)
    h2, el2, er2 = _mid(acc1.reshape(NC, NPAD, D), den1.reshape(NC, NPAD),
                        b1.reshape(1, D), W2,
                        al2.reshape(1, D), ar2.reshape(1, D))
    acc2, den2 = _edge(h2, el2.reshape(NPAD), er2.reshape(NPAD), src, dst)
    out = _fin(acc2.reshape(NC, NPAD, D), den2.reshape(NC, NPAD),
               b2.reshape(1, D))
    return out[:N]


# speedup vs baseline: 57.1526x; 1.0253x over previous
import functools

import jax
import jax.numpy as jnp
from jax import lax
from jax.experimental import pallas as pl
from jax.experimental.pallas import tpu as pltpu
from jax.experimental.pallas import tpu_sc as plsc

N = 10000
NPAD = 10240
E = 320000
D = 128

NC = 2
NS = 16
NW = NC * NS
EPT = E // NW
B = 80
NB = EPT // B
NBUF = 4
RPT = NPAD // NS
RBLK = 2048
GRID = NPAD // RBLK


def _proj_body(x_ref, w_ref, al_ref, ar_ref, h_ref, el_ref, er_ref):
    h = jnp.dot(x_ref[...], w_ref[...], preferred_element_type=jnp.float32)
    h_ref[...] = h
    el_ref[...] = jnp.sum(h * al_ref[...], axis=1)[None, :]
    er_ref[...] = jnp.sum(h * ar_ref[...], axis=1)[None, :]


_proj = pl.pallas_call(
    _proj_body,
    grid=(GRID,),
    in_specs=[
        pl.BlockSpec((RBLK, D), lambda i: (i, 0)),
        pl.BlockSpec((D, D), lambda i: (0, 0)),
        pl.BlockSpec((1, D), lambda i: (0, 0)),
        pl.BlockSpec((1, D), lambda i: (0, 0)),
    ],
    out_specs=[
        pl.BlockSpec((RBLK, D), lambda i: (i, 0)),
        pl.BlockSpec((1, RBLK), lambda i: (0, i)),
        pl.BlockSpec((1, RBLK), lambda i: (0, i)),
    ],
    out_shape=[
        jax.ShapeDtypeStruct((NPAD, D), jnp.float32),
        jax.ShapeDtypeStruct((1, NPAD), jnp.float32),
        jax.ShapeDtypeStruct((1, NPAD), jnp.float32),
    ],
)


def _mid_body(acc_ref, den_ref, b_ref, w_ref, al_ref, ar_ref,
              h_ref, el_ref, er_ref):
    acc = acc_ref[0] + acc_ref[1]
    den = den_ref[0] + den_ref[1]
    hin = acc / (den[:, None] + 1e-9) + b_ref[...]
    hin = jnp.maximum(hin, 0.0)
    h = jnp.dot(hin, w_ref[...], preferred_element_type=jnp.float32)
    h_ref[...] = h
    el_ref[...] = jnp.sum(h * al_ref[...], axis=1)[None, :]
    er_ref[...] = jnp.sum(h * ar_ref[...], axis=1)[None, :]


_mid = pl.pallas_call(
    _mid_body,
    grid=(GRID,),
    in_specs=[
        pl.BlockSpec((NC, RBLK, D), lambda i: (0, i, 0)),
        pl.BlockSpec((NC, RBLK), lambda i: (0, i)),
        pl.BlockSpec((1, D), lambda i: (0, 0)),
        pl.BlockSpec((D, D), lambda i: (0, 0)),
        pl.BlockSpec((1, D), lambda i: (0, 0)),
        pl.BlockSpec((1, D), lambda i: (0, 0)),
    ],
    out_specs=[
        pl.BlockSpec((RBLK, D), lambda i: (i, 0)),
        pl.BlockSpec((1, RBLK), lambda i: (0, i)),
        pl.BlockSpec((1, RBLK), lambda i: (0, i)),
    ],
    out_shape=[
        jax.ShapeDtypeStruct((NPAD, D), jnp.float32),
        jax.ShapeDtypeStruct((1, NPAD), jnp.float32),
        jax.ShapeDtypeStruct((1, NPAD), jnp.float32),
    ],
)


def _fin_body(acc_ref, den_ref, b_ref, o_ref):
    acc = acc_ref[0] + acc_ref[1]
    den = den_ref[0] + den_ref[1]
    o_ref[...] = acc / (den[:, None] + 1e-9) + b_ref[...]


_fin = pl.pallas_call(
    _fin_body,
    grid=(GRID,),
    in_specs=[
        pl.BlockSpec((NC, RBLK, D), lambda i: (0, i, 0)),
        pl.BlockSpec((NC, RBLK), lambda i: (0, i)),
        pl.BlockSpec((1, D), lambda i: (0, 0)),
    ],
    out_specs=pl.BlockSpec((RBLK, D), lambda i: (i, 0)),
    out_shape=jax.ShapeDtypeStruct((NPAD, D), jnp.float32),
)


_mesh = plsc.VectorSubcoreMesh(
    core_axis_name="c", subcore_axis_name="s", num_cores=NC, num_subcores=NS)


@functools.partial(
    pl.kernel,
    out_type=[
        jax.ShapeDtypeStruct((NC * NPAD, D), jnp.float32),
        jax.ShapeDtypeStruct((NC * NPAD,), jnp.float32),
    ],
    mesh=_mesh,
    compiler_params=pltpu.CompilerParams(needs_layout_passes=False),
    scratch_types=(
        [pltpu.VMEM((B,), jnp.int32) for _ in range(NBUF)]
        + [pltpu.VMEM((B,), jnp.int32) for _ in range(NBUF)]
        + [pltpu.VMEM((B,), jnp.float32) for _ in range(NBUF)]
        + [pltpu.VMEM((B,), jnp.float32) for _ in range(NBUF)]
        + [pltpu.VMEM((B,), jnp.float32) for _ in range(NBUF)]
        + [pltpu.VMEM((B, D), jnp.float32) for _ in range(NBUF)]
        + [
            pltpu.VMEM((RPT,), jnp.float32),
            pltpu.VMEM_SHARED((NPAD, D), jnp.float32),
            pltpu.VMEM_SHARED((NPAD,), jnp.float32),
        ]
        + [pltpu.SemaphoreType.DMA for _ in range(3 * NBUF)]
    ),
)
def _edge(h_hbm, el_hbm, er_hbm, src_hbm, dst_hbm, acc_out, den_out,
          *scratch):
    srcs = scratch[0:NBUF]
    dsts = scratch[NBUF:2 * NBUF]
    elbs = scratch[2 * NBUF:3 * NBUF]
    erbs = scratch[3 * NBUF:4 * NBUF]
    ees = scratch[4 * NBUF:5 * NBUF]
    rows = scratch[5 * NBUF:6 * NBUF]
    zer_v = scratch[6 * NBUF]
    acc_sh = scratch[6 * NBUF + 1]
    den_sh = scratch[6 * NBUF + 2]
    semA = scratch[6 * NBUF + 3:6 * NBUF + 3 + NBUF]
    semR = scratch[6 * NBUF + 3 + NBUF:6 * NBUF + 3 + 2 * NBUF]
    semS = scratch[6 * NBUF + 3 + 2 * NBUF:6 * NBUF + 3 + 3 * NBUF]

    c = lax.axis_index("c")
    s = lax.axis_index("s")
    wid = c * NS + s
    ebase = wid * EPT

    zero16 = jnp.zeros((16,), jnp.float32)

    def _zrow(r, carry):
        for j in range(D // 16):
            rows[0][r, pl.ds(j * 16, 16)] = zero16
        return carry

    lax.fori_loop(0, B, _zrow, 0)

    def _zv(i, carry):
        zer_v[pl.ds(i * 16, 16)] = zero16
        return carry

    lax.fori_loop(0, RPT // 16, _zv, 0)

    def _zacc(i, carry):
        pltpu.sync_copy(rows[0], acc_sh.at[pl.ds(s * RPT + i * B, B)])
        return carry

    lax.fori_loop(0, RPT // B, _zacc, 0)
    pltpu.sync_copy(zer_v, den_sh.at[pl.ds(s * RPT, RPT)])
    plsc.subcore_barrier()

    def _issue_idx(b, p):
        off = ebase + lax.rem(b, NB) * B
        pltpu.async_copy(src_hbm.at[pl.ds(off, B)], srcs[p], semA[p])
        pltpu.async_copy(dst_hbm.at[pl.ds(off, B)], dsts[p], semA[p])

    def _wait_idx(p):
        pltpu.make_async_copy(
            src_hbm.at[pl.ds(ebase, B)], srcs[p], semA[p]).wait()
        pltpu.make_async_copy(
            dst_hbm.at[pl.ds(ebase, B)], dsts[p], semA[p]).wait()

    def _issue_gathers(p):
        pltpu.async_copy(h_hbm.at[srcs[p]], rows[p], semR[p])
        pltpu.async_copy(el_hbm.at[srcs[p]], elbs[p], semA[p])
        pltpu.async_copy(er_hbm.at[dsts[p]], erbs[p], semA[p])

    def _wait_meta(p):
        pltpu.make_async_copy(
            el_hbm.at[srcs[p]], elbs[p], semA[p]).wait()
        pltpu.make_async_copy(
            er_hbm.at[dsts[p]], erbs[p], semA[p]).wait()

    def _wait_rows(p):
        pltpu.make_async_copy(h_hbm.at[srcs[p]], rows[p], semR[p]).wait()

    def _wait_scat(p):
        pltpu.make_async_copy(ees[p], den_sh.at[dsts[p]], semS[p]).wait()
        pltpu.make_async_copy(rows[p], acc_sh.at[dsts[p]], semS[p]).wait()

    def _phead(p):
        _wait_meta(p)

        def _grp(g, carry2):
            ev = elbs[p][pl.ds(g * 16, 16)] + erbs[p][pl.ds(g * 16, 16)]
            ev = jnp.where(ev >= 0.0, ev, ev * jnp.float32(0.2))
            ees[p][pl.ds(g * 16, 16)] = jnp.exp(ev)
            return carry2

        lax.fori_loop(0, B // 16, _grp, 0)
        pltpu.async_copy(ees[p], den_sh.at[dsts[p]], semS[p], add=True)

    def _ptail(p):
        _wait_rows(p)

        def _scale(g, carry2):
            ee16 = ees[p][pl.ds(g * 16, 16)]
            for k in range(16):
                sc = ee16[k]
                r = g * 16 + k
                for j in range(D // 16):
                    rows[p][r, pl.ds(j * 16, 16)] = (
                        rows[p][r, pl.ds(j * 16, 16)] * sc)
            return carry2

        lax.fori_loop(0, B // 16, _scale, 0)
        pltpu.async_copy(rows[p], acc_sh.at[dsts[p]], semS[p], add=True)

    def _s1(b, p):
        _wait_scat(p)
        _issue_idx(b, p)

    def _s2(p):
        _wait_idx(p)
        _issue_gathers(p)

    for p in range(NBUF - 1):
        _issue_idx(p, p)
    for p in range(NBUF - 2):
        _wait_idx(p)
        _issue_gathers(p)

    for k in range(NBUF):
        _phead(k)
        _s2((k + 2) % NBUF)
        _ptail(k)
        if k == 0:
            _issue_idx(NBUF - 1, NBUF - 1)
        else:
            _s1(k - 1 + NBUF, k - 1)

    @pl.loop(NBUF, (NB - 1) // NBUF * NBUF, step=NBUF)
    def _body(g):
        for k in range(NBUF):
            _phead(k)
            _s2((k + 2) % NBUF)
            _ptail(k)
            _s1(g + k + NBUF - 1, (k - 1) % NBUF)

    _phead(0)
    _ptail(0)
    _wait_scat(0)
    _wait_scat(NBUF - 1)
    _wait_meta(1)
    _wait_rows(1)
    _wait_idx(2)
    plsc.subcore_barrier()

    obase = c * NPAD + s * RPT
    pltpu.sync_copy(acc_sh.at[pl.ds(s * RPT, RPT)],
                    acc_out.at[pl.ds(obase, RPT)])
    pltpu.sync_copy(den_sh.at[pl.ds(s * RPT, RPT)],
                    den_out.at[pl.ds(obase, RPT)])


def kernel(x, edge_index, W1, al1, ar1, b1, W2, al2, ar2, b2):
    src = edge_index[0].astype(jnp.int32)
    dst = edge_index[1].astype(jnp.int32)
    xp = jnp.pad(x, ((0, NPAD - N), (0, 0)))

    h1, el1, er1 = _proj(xp, W1, al1.reshape(1, D), ar1.reshape(1, D))
    acc1, den1 = _edge(h1, el1.reshape(NPAD), er1.reshape(NPAD), src, dst)
    h2, el2, er2 = _mid(acc1.reshape(NC, NPAD, D), den1.reshape(NC, NPAD),
                        b1.reshape(1, D), W2,
                        al2.reshape(1, D), ar2.reshape(1, D))
    acc2, den2 = _edge(h2, el2.reshape(NPAD), er2.reshape(NPAD), src, dst)
    out = _fin(acc2.reshape(NC, NPAD, D), den2.reshape(NC, NPAD),
               b2.reshape(1, D))
    return out[:N]
